# baseline (speedup 1.0000x reference)
import functools

import jax
import jax.numpy as jnp
from jax import lax
from jax.experimental import pallas as pl
from jax.experimental.pallas import tpu as pltpu
from jax.experimental.pallas import tpu_sc as plsc

NW = 32
L = 16
C = 16384
U = 8

N_BOND = 2_000_000
N_ANGLE = 2_000_000
N_DIHEDRAL = 2_000_000
N_ATOM = 1_000_000

OFF_BK = 0
OFF_B0 = 112
OFF_AK = 224
OFF_T0 = 336
OFF_DP0 = 448
OFF_DP1 = 656
OFF_DP2 = 864
OFF_DP3 = 1072
OFF_DP4 = 1280
OFF_ES = 1488
OFF_EPS = 1552
OFF_Q = 1616
OFF_BP = 1680
OFF_AP = 1792
TBL_LEN = 1904
_BK_SCALE = 128.0
_B0_SCALE = 16384.0
_AK_SCALE = 512.0
_T0_SCALE = 8192.0

_COS_COEFFS = (-1.26237509e-3, 4.14567630e-2, -4.99885707e-1, 9.99989971e-1)
_INV_PI = 0.3183098861837907
_PI_F32 = 3.14159274
_MAGIC = 12582912.0


def _cos(phi):
    t = phi * _INV_PI
    m = t + _MAGIC
    kf = m - _MAGIC
    r = phi - kf * _PI_F32
    x2 = r * r
    p = jnp.full((L,), _COS_COEFFS[0], dtype=jnp.float32)
    for c in _COS_COEFFS[1:]:
        p = p * x2 + c
    sign = (plsc.bitcast(m, jnp.int32) & 1) << 31
    return plsc.bitcast(plsc.bitcast(p, jnp.int32) ^ sign, jnp.float32)


def _shard_bounds(wid, n_elems):
    n_v = n_elems // L
    base, rem = n_v // NW, n_v % NW
    start_v = base * wid + jnp.minimum(wid, rem)
    cnt_v = base + jnp.where(wid < rem, 1, 0)
    return start_v * L, (start_v + cnt_v) * L


def _tree_add(a, b):
    return jax.tree.map(lambda x, y: x + y, a, b)


def _ff_body(bt_hbm, bb_hbm, at_hbm, av_hbm, dt_hbm, dv_hbm, nt_hbm,
             kb_hbm, b0_hbm, ka_hbm, t0_hbm, k1_hbm, k2_hbm, k3_hbm, k4_hbm,
             sg_hbm, ep_hbm, q_hbm,
             out_hbm, ti0, tf0, ti1, tf1, tbl, ov, sem0, sem1):
    wid = lax.axis_index("s") * 2 + lax.axis_index("c")

    for src, off, n in ((kb_hbm, OFF_BK, 100), (b0_hbm, OFF_B0, 100),
                        (ka_hbm, OFF_AK, 100), (t0_hbm, OFF_T0, 100),
                        (k1_hbm, OFF_DP1, 200), (k2_hbm, OFF_DP2, 200),
                        (k3_hbm, OFF_DP3, 200), (k4_hbm, OFF_DP4, 200),
                        (sg_hbm, OFF_ES, 50), (ep_hbm, OFF_EPS, 50),
                        (q_hbm, OFF_Q, 50)):
        pltpu.sync_copy(src, tbl.at[pl.ds(off, n)])
    for v in range(7):
        i = v * L
        x = tbl[pl.ds(OFF_BK + i, L)]
        tbl[pl.ds(OFF_BK + i, L)] = (x * 10.0) * (x * 10.0)
        x = tbl[pl.ds(OFF_B0 + i, L)]
        tbl[pl.ds(OFF_B0 + i, L)] = x * x
        x = tbl[pl.ds(OFF_AK + i, L)]
        tbl[pl.ds(OFF_AK + i, L)] = (x * 10.0) * (x * 10.0)
        x = tbl[pl.ds(OFF_T0 + i, L)]
        tbl[pl.ds(OFF_T0 + i, L)] = x * x
    for v in range(13):
        i = v * L
        b1 = tbl[pl.ds(OFF_DP1 + i, L)] * 10.0
        b2 = tbl[pl.ds(OFF_DP2 + i, L)] * 10.0
        b3 = tbl[pl.ds(OFF_DP3 + i, L)] * 10.0
        b4 = tbl[pl.ds(OFF_DP4 + i, L)] * 10.0
        tbl[pl.ds(OFF_DP0 + i, L)] = b1 + 2.0 * b2 + b3
        tbl[pl.ds(OFF_DP1 + i, L)] = b1 - 3.0 * b3
        tbl[pl.ds(OFF_DP2 + i, L)] = 8.0 * b4 - 2.0 * b2
        tbl[pl.ds(OFF_DP3 + i, L)] = 4.0 * b3
        tbl[pl.ds(OFF_DP4 + i, L)] = -8.0 * b4
    for v in range(4):
        i = v * L
        tbl[pl.ds(OFF_ES + i, L)] = (tbl[pl.ds(OFF_ES + i, L)]
                                     * tbl[pl.ds(OFF_EPS + i, L)])
    for v in range(7):
        i = v * L
        hi = (tbl[pl.ds(OFF_BK + i, L)] * _BK_SCALE + 0.5).astype(jnp.int32)
        lo = (tbl[pl.ds(OFF_B0 + i, L)] * _B0_SCALE + 0.5).astype(jnp.int32)
        tbl[pl.ds(OFF_BP + i, L)] = plsc.bitcast((hi << 16) | lo, jnp.float32)
        hi = (tbl[pl.ds(OFF_AK + i, L)] * _AK_SCALE + 0.5).astype(jnp.int32)
        lo = (tbl[pl.ds(OFF_T0 + i, L)] * _T0_SCALE + 0.5).astype(jnp.int32)
        tbl[pl.ds(OFF_AP + i, L)] = plsc.bitcast((hi << 16) | lo, jnp.float32)

    t_bp = tbl.at[pl.ds(OFF_BP, 112)]
    t_ap = tbl.at[pl.ds(OFF_AP, 112)]
    t_p0 = tbl.at[pl.ds(OFF_DP0, 208)]
    t_p1 = tbl.at[pl.ds(OFF_DP1, 208)]
    t_p2 = tbl.at[pl.ds(OFF_DP2, 208)]
    t_p3 = tbl.at[pl.ds(OFF_DP3, 208)]
    t_p4 = tbl.at[pl.ds(OFF_DP4, 208)]
    t_es = tbl.at[pl.ds(OFF_ES, 64)]
    t_q = tbl.at[pl.ds(OFF_Q, 64)]

    def bond_body(ty, b):
        w = plsc.bitcast(plsc.load_gather(t_bp, [ty]), jnp.int32)
        kq = lax.shift_right_logical(w, 16).astype(jnp.float32)
        b0s = (w & 0xFFFF).astype(jnp.float32) * (1.0 / _B0_SCALE)
        d = b - b0s
        return kq * d * d

    def angle_body(ty, th):
        w = plsc.bitcast(plsc.load_gather(t_ap, [ty]), jnp.int32)
        kq = lax.shift_right_logical(w, 16).astype(jnp.float32)
        t0s = (w & 0xFFFF).astype(jnp.float32) * (1.0 / _T0_SCALE)
        d = th - t0s
        return kq * d * d

    def dih_body(ty, phi):
        p0 = plsc.load_gather(t_p0, [ty])
        p1 = plsc.load_gather(t_p1, [ty])
        p2 = plsc.load_gather(t_p2, [ty])
        p3 = plsc.load_gather(t_p3, [ty])
        p4 = plsc.load_gather(t_p4, [ty])
        c = _cos(phi)
        return (((p4 * c + p3) * c + p2) * c + p1) * c + p0

    def atom_body(ty, _):
        es = plsc.load_gather(t_es, [ty])
        q = plsc.load_gather(t_q, [ty])
        return (es, q)

    n_big = -(-((N_BOND // L // NW + 1) * L) // C)
    n_atom = -(-((N_ATOM // L // NW + 1) * L) // C)
    stages = []
    for key, ty_ref, val_ref, n_total, n_ch, body in (
            ("b", bt_hbm, bb_hbm, N_BOND, n_big, bond_body),
            ("a", at_hbm, av_hbm, N_ANGLE, n_big, angle_body),
            ("d", dt_hbm, dv_hbm, N_DIHEDRAL, n_big, dih_body),
            ("n", nt_hbm, None, N_ATOM, n_atom, atom_body)):
        g0, g1 = _shard_bounds(wid, n_total)
        for j in range(n_ch):
            stages.append(dict(key=key, ty=ty_ref, val=val_ref, n=n_total,
                               g0=g0, g1=g1, j=j, body=body,
                               last=(j == n_ch - 1)))

    bufs = ((ti0, tf0, sem0), (ti1, tf1, sem1))

    def issue(st, buf):
        ti, tf, sem = buf
        s = jnp.minimum(st["g0"] + st["j"] * C, st["n"] - C)
        hs = [pltpu.async_copy(st["ty"].at[pl.ds(s, C)], ti, sem)]
        if st["val"] is not None:
            hs.append(pltpu.async_copy(st["val"].at[pl.ds(s, C)], tf, sem))
        return hs

    def compute(st, buf, acc):
        ti, tf, _ = buf
        has_val = st["val"] is not None
        body = st["body"]

        def group(base_v, acc):
            es = []
            for u in range(U):
                o = (base_v + u) * L
                ty = ti[pl.ds(o, L)]
                val = tf[pl.ds(o, L)] if has_val else None
                es.append(body(ty, val))
            while len(es) > 1:
                es = [_tree_add(es[k], es[k + 1]) for k in range(0, len(es), 2)]
            return _tree_add(acc, es[0])

        if not st["last"]:
            return lax.fori_loop(0, C // L // U,
                                 lambda it, a: group(it * U, a), acc)
        e_start = st["g0"] + st["j"] * C
        s = jnp.minimum(e_start, st["n"] - C)
        e_end = jnp.minimum(e_start + C, st["g1"])
        v_lo = (e_start - s) // L
        v_hi = (e_end - s) // L
        n_main = (v_hi - v_lo) // U
        acc = lax.fori_loop(0, n_main,
                            lambda it, a: group(v_lo + it * U, a), acc)

        def rem(v, a):
            o = v * L
            ty = ti[pl.ds(o, L)]
            val = tf[pl.ds(o, L)] if has_val else None
            return _tree_add(a, body(ty, val))

        return lax.fori_loop(v_lo + n_main * U, v_hi, rem, acc)

    zero = jnp.zeros((L,), dtype=jnp.float32)
    accs = {"b": zero, "a": zero, "d": zero, "n": (zero, zero)}

    pend = issue(stages[0], bufs[0])
    for i, st in enumerate(stages):
        nxt = issue(stages[i + 1], bufs[(i + 1) % 2]) if i + 1 < len(stages) else None
        for h in pend:
            h.wait()
        accs[st["key"]] = compute(st, bufs[i % 2], accs[st["key"]])
        pend = nxt

    ov[pl.ds(0, L)] = (accs["b"] * (1.0 / _BK_SCALE)
                       + accs["a"] * (1.0 / _AK_SCALE)
                       + accs["d"] + accs["n"][0])
    ov[pl.ds(L, L)] = accs["n"][1]
    pltpu.sync_copy(ov, out_hbm.at[wid])


_ff = functools.partial(
    pl.kernel,
    out_type=jax.ShapeDtypeStruct((NW, 2 * L), jnp.float32),
    mesh=plsc.VectorSubcoreMesh(core_axis_name="c", subcore_axis_name="s"),
    compiler_params=pltpu.CompilerParams(needs_layout_passes=False),
    scratch_types=[
        pltpu.VMEM((C,), jnp.int32),
        pltpu.VMEM((C,), jnp.float32),
        pltpu.VMEM((C,), jnp.int32),
        pltpu.VMEM((C,), jnp.float32),
        pltpu.VMEM((TBL_LEN,), jnp.float32),
        pltpu.VMEM((2 * L,), jnp.float32),
        pltpu.SemaphoreType.DMA,
        pltpu.SemaphoreType.DMA,
    ],
)(_ff_body)


def kernel(bond_type, bond_b, angle_type, angle_theta, dihedral_type,
           dihedral_phi, node_type, k_bond, b0, k_angle, theta0,
           k1, k2, k3, k4, sigma, epsilon, charge):
    parts = _ff(bond_type.astype(jnp.int32), bond_b,
                angle_type.astype(jnp.int32), angle_theta,
                dihedral_type.astype(jnp.int32), dihedral_phi,
                node_type.astype(jnp.int32),
                k_bond, b0, k_angle, theta0, k1, k2, k3, k4,
                sigma, epsilon, charge)
    e = jnp.sum(parts[:, :L])
    q = jnp.sum(parts[:, L:])
    return e + q * q


# speedup vs baseline: 1.0850x; 1.0850x over previous
import functools

import jax
import jax.numpy as jnp
from jax import lax
from jax.experimental import pallas as pl
from jax.experimental.pallas import tpu as pltpu
from jax.experimental.pallas import tpu_sc as plsc

NW = 32
L = 16
C = 16384
U = 8

N_BOND = 2_000_000
N_ANGLE = 2_000_000
N_DIHEDRAL = 2_000_000
N_ATOM = 1_000_000

OFF_BK = 0
OFF_B0 = 112
OFF_AK = 224
OFF_T0 = 336
OFF_DP0 = 448
OFF_DP1 = 656
OFF_DP2 = 864
OFF_DP3 = 1072
OFF_DP4 = 1280
OFF_ES = 1488
OFF_EPS = 1552
OFF_Q = 1616
OFF_BP = 1680
OFF_AP = 1792
TBL_LEN = 1904
_BK_SCALE = 128.0
_B0_SCALE = 16384.0
_AK_SCALE = 512.0
_T0_SCALE = 8192.0

_COS_COEFFS = (-1.26237509e-3, 4.14567630e-2, -4.99885707e-1, 9.99989971e-1)
_INV_PI = 0.3183098861837907
_PI_F32 = 3.14159274
_MAGIC = 12582912.0


def _cos(phi):
    t = phi * _INV_PI
    m = t + _MAGIC
    kf = m - _MAGIC
    r = phi - kf * _PI_F32
    x2 = r * r
    p = jnp.full((L,), _COS_COEFFS[0], dtype=jnp.float32)
    for c in _COS_COEFFS[1:]:
        p = p * x2 + c
    sign = (plsc.bitcast(m, jnp.int32) & 1) << 31
    return plsc.bitcast(plsc.bitcast(p, jnp.int32) ^ sign, jnp.float32)


def _shard_bounds(wid, n_elems):
    n_v = n_elems // L
    base, rem = n_v // NW, n_v % NW
    start_v = base * wid + jnp.minimum(wid, rem)
    cnt_v = base + jnp.where(wid < rem, 1, 0)
    return start_v * L, (start_v + cnt_v) * L


def _tree_add(a, b):
    return jax.tree.map(lambda x, y: x + y, a, b)


def _ff_body(bt_hbm, bb_hbm, at_hbm, av_hbm, dt_hbm, dv_hbm, nt_hbm,
             kb_hbm, b0_hbm, ka_hbm, t0_hbm, k1_hbm, k2_hbm, k3_hbm, k4_hbm,
             sg_hbm, ep_hbm, q_hbm,
             out_hbm, ti0, tf0, ti1, tf1, tbl, ov, sem0, sem1):
    wid = lax.axis_index("s") * 2 + lax.axis_index("c")

    tbl_copies = [
        pltpu.async_copy(src, tbl.at[pl.ds(off, n)], sem1)
        for src, off, n in ((kb_hbm, OFF_BK, 100), (b0_hbm, OFF_B0, 100),
                            (ka_hbm, OFF_AK, 100), (t0_hbm, OFF_T0, 100),
                            (k1_hbm, OFF_DP1, 200), (k2_hbm, OFF_DP2, 200),
                            (k3_hbm, OFF_DP3, 200), (k4_hbm, OFF_DP4, 200),
                            (sg_hbm, OFF_ES, 50), (ep_hbm, OFF_EPS, 50),
                            (q_hbm, OFF_Q, 50))]

    def _transform_table():
        for h in tbl_copies:
            h.wait()
        for v in range(7):
            i = v * L
            x = tbl[pl.ds(OFF_BK + i, L)]
            tbl[pl.ds(OFF_BK + i, L)] = (x * 10.0) * (x * 10.0)
            x = tbl[pl.ds(OFF_B0 + i, L)]
            tbl[pl.ds(OFF_B0 + i, L)] = x * x
            x = tbl[pl.ds(OFF_AK + i, L)]
            tbl[pl.ds(OFF_AK + i, L)] = (x * 10.0) * (x * 10.0)
            x = tbl[pl.ds(OFF_T0 + i, L)]
            tbl[pl.ds(OFF_T0 + i, L)] = x * x
        for v in range(13):
            i = v * L
            b1 = tbl[pl.ds(OFF_DP1 + i, L)] * 10.0
            b2 = tbl[pl.ds(OFF_DP2 + i, L)] * 10.0
            b3 = tbl[pl.ds(OFF_DP3 + i, L)] * 10.0
            b4 = tbl[pl.ds(OFF_DP4 + i, L)] * 10.0
            tbl[pl.ds(OFF_DP0 + i, L)] = b1 + 2.0 * b2 + b3
            tbl[pl.ds(OFF_DP1 + i, L)] = b1 - 3.0 * b3
            tbl[pl.ds(OFF_DP2 + i, L)] = 8.0 * b4 - 2.0 * b2
            tbl[pl.ds(OFF_DP3 + i, L)] = 4.0 * b3
            tbl[pl.ds(OFF_DP4 + i, L)] = -8.0 * b4
        for v in range(4):
            i = v * L
            tbl[pl.ds(OFF_ES + i, L)] = (tbl[pl.ds(OFF_ES + i, L)]
                                         * tbl[pl.ds(OFF_EPS + i, L)])
        for v in range(7):
            i = v * L
            hi = (tbl[pl.ds(OFF_BK + i, L)] * _BK_SCALE + 0.5).astype(jnp.int32)
            lo = (tbl[pl.ds(OFF_B0 + i, L)] * _B0_SCALE + 0.5).astype(jnp.int32)
            tbl[pl.ds(OFF_BP + i, L)] = plsc.bitcast((hi << 16) | lo,
                                                     jnp.float32)
            hi = (tbl[pl.ds(OFF_AK + i, L)] * _AK_SCALE + 0.5).astype(jnp.int32)
            lo = (tbl[pl.ds(OFF_T0 + i, L)] * _T0_SCALE + 0.5).astype(jnp.int32)
            tbl[pl.ds(OFF_AP + i, L)] = plsc.bitcast((hi << 16) | lo,
                                                     jnp.float32)

    t_bp = tbl.at[pl.ds(OFF_BP, 112)]
    t_ap = tbl.at[pl.ds(OFF_AP, 112)]
    t_p0 = tbl.at[pl.ds(OFF_DP0, 208)]
    t_p1 = tbl.at[pl.ds(OFF_DP1, 208)]
    t_p2 = tbl.at[pl.ds(OFF_DP2, 208)]
    t_p3 = tbl.at[pl.ds(OFF_DP3, 208)]
    t_p4 = tbl.at[pl.ds(OFF_DP4, 208)]
    t_es = tbl.at[pl.ds(OFF_ES, 64)]
    t_q = tbl.at[pl.ds(OFF_Q, 64)]

    def bond_body(ty, b):
        w = plsc.bitcast(plsc.load_gather(t_bp, [ty]), jnp.int32)
        kq = lax.shift_right_logical(w, 16).astype(jnp.float32)
        b0s = (w & 0xFFFF).astype(jnp.float32) * (1.0 / _B0_SCALE)
        d = b - b0s
        return kq * d * d

    def angle_body(ty, th):
        w = plsc.bitcast(plsc.load_gather(t_ap, [ty]), jnp.int32)
        kq = lax.shift_right_logical(w, 16).astype(jnp.float32)
        t0s = (w & 0xFFFF).astype(jnp.float32) * (1.0 / _T0_SCALE)
        d = th - t0s
        return kq * d * d

    def dih_body(ty, phi):
        p0 = plsc.load_gather(t_p0, [ty])
        p1 = plsc.load_gather(t_p1, [ty])
        p2 = plsc.load_gather(t_p2, [ty])
        p3 = plsc.load_gather(t_p3, [ty])
        p4 = plsc.load_gather(t_p4, [ty])
        c = _cos(phi)
        return (((p4 * c + p3) * c + p2) * c + p1) * c + p0

    def atom_body(ty, _):
        es = plsc.load_gather(t_es, [ty])
        q = plsc.load_gather(t_q, [ty])
        return (es, q)

    n_big = -(-((N_BOND // L // NW + 1) * L) // C)
    n_atom = -(-((N_ATOM // L // NW + 1) * L) // C)
    stages = []
    for key, ty_ref, val_ref, n_total, n_ch, body in (
            ("b", bt_hbm, bb_hbm, N_BOND, n_big, bond_body),
            ("a", at_hbm, av_hbm, N_ANGLE, n_big, angle_body),
            ("d", dt_hbm, dv_hbm, N_DIHEDRAL, n_big, dih_body),
            ("n", nt_hbm, None, N_ATOM, n_atom, atom_body)):
        g0, g1 = _shard_bounds(wid, n_total)
        for j in range(n_ch):
            stages.append(dict(key=key, ty=ty_ref, val=val_ref, n=n_total,
                               g0=g0, g1=g1, j=j, body=body,
                               last=(j == n_ch - 1)))

    bufs = ((ti0, tf0, sem0), (ti1, tf1, sem1))

    def issue(st, buf):
        ti, tf, sem = buf
        s = jnp.minimum(st["g0"] + st["j"] * C, st["n"] - C)
        hs = [pltpu.async_copy(st["ty"].at[pl.ds(s, C)], ti, sem)]
        if st["val"] is not None:
            hs.append(pltpu.async_copy(st["val"].at[pl.ds(s, C)], tf, sem))
        return hs

    def compute(st, buf, acc):
        ti, tf, _ = buf
        has_val = st["val"] is not None
        body = st["body"]

        def group(base_v, acc):
            es = []
            for u in range(U):
                o = (base_v + u) * L
                ty = ti[pl.ds(o, L)]
                val = tf[pl.ds(o, L)] if has_val else None
                es.append(body(ty, val))
            while len(es) > 1:
                es = [_tree_add(es[k], es[k + 1]) for k in range(0, len(es), 2)]
            return _tree_add(acc, es[0])

        if not st["last"]:
            return lax.fori_loop(0, C // L // U,
                                 lambda it, a: group(it * U, a), acc)
        e_start = st["g0"] + st["j"] * C
        s = jnp.minimum(e_start, st["n"] - C)
        e_end = jnp.minimum(e_start + C, st["g1"])
        v_lo = (e_start - s) // L
        v_hi = (e_end - s) // L
        n_main = (v_hi - v_lo) // U
        acc = lax.fori_loop(0, n_main,
                            lambda it, a: group(v_lo + it * U, a), acc)

        def rem(v, a):
            o = v * L
            ty = ti[pl.ds(o, L)]
            val = tf[pl.ds(o, L)] if has_val else None
            return _tree_add(a, body(ty, val))

        return lax.fori_loop(v_lo + n_main * U, v_hi, rem, acc)

    zero = jnp.zeros((L,), dtype=jnp.float32)
    accs = {"b": zero, "a": zero, "d": zero, "n": (zero, zero)}

    pend = issue(stages[0], bufs[0])
    _transform_table()
    for i, st in enumerate(stages):
        nxt = issue(stages[i + 1], bufs[(i + 1) % 2]) if i + 1 < len(stages) else None
        for h in pend:
            h.wait()
        accs[st["key"]] = compute(st, bufs[i % 2], accs[st["key"]])
        pend = nxt

    ov[pl.ds(0, L)] = (accs["b"] * (1.0 / _BK_SCALE)
                       + accs["a"] * (1.0 / _AK_SCALE)
                       + accs["d"] + accs["n"][0])
    ov[pl.ds(L, L)] = accs["n"][1]
    pltpu.sync_copy(ov, out_hbm.at[wid])


_ff = functools.partial(
    pl.kernel,
    out_type=jax.ShapeDtypeStruct((NW, 2 * L), jnp.float32),
    mesh=plsc.VectorSubcoreMesh(core_axis_name="c", subcore_axis_name="s"),
    compiler_params=pltpu.CompilerParams(needs_layout_passes=False),
    scratch_types=[
        pltpu.VMEM((C,), jnp.int32),
        pltpu.VMEM((C,), jnp.float32),
        pltpu.VMEM((C,), jnp.int32),
        pltpu.VMEM((C,), jnp.float32),
        pltpu.VMEM((TBL_LEN,), jnp.float32),
        pltpu.VMEM((2 * L,), jnp.float32),
        pltpu.SemaphoreType.DMA,
        pltpu.SemaphoreType.DMA,
    ],
)(_ff_body)


def kernel(bond_type, bond_b, angle_type, angle_theta, dihedral_type,
           dihedral_phi, node_type, k_bond, b0, k_angle, theta0,
           k1, k2, k3, k4, sigma, epsilon, charge):
    parts = _ff(bond_type.astype(jnp.int32), bond_b,
                angle_type.astype(jnp.int32), angle_theta,
                dihedral_type.astype(jnp.int32), dihedral_phi,
                node_type.astype(jnp.int32),
                k_bond, b0, k_angle, theta0, k1, k2, k3, k4,
                sigma, epsilon, charge)
    e = jnp.sum(parts[:, :L])
    q = jnp.sum(parts[:, L:])
    return e + q * q
